# baseline (speedup 1.0000x reference)
import functools

import jax
import jax.numpy as jnp
import numpy as np
from jax import lax
from jax.experimental import pallas as pl
from jax.experimental.pallas import tpu as pltpu
from jax.experimental.pallas import tpu_sc as plsc

LATENT = 64
ENT = 1000000
REL = 1000
BATCH = 1024
TWOB = 2 * BATCH
LMBDA = 0.1
_LOG2 = float(np.log(2.0))

_info = plsc.get_sparse_core_info()
_NC, _NS, _L = _info.num_cores, _info.num_subcores, _info.num_lanes
_NW = _NC * _NS
_BPW = TWOB // _NW
_NG = _BPW // _L


def _score_body(ent_hbm, rel_hbm, bh_hbm, bt_hbm, br_hbm,
                s_out, ssq_out,
                idxh_v, idxt_v, idxr_v, fh_v, ft_v, fr_v,
                gh_v, gt_v, gr_v, s_v, ssq_v, sem_h, sem_t, sem_r):
    wid = lax.axis_index("s") * _NC + lax.axis_index("c")
    base = wid * _BPW
    pltpu.sync_copy(bh_hbm.at[pl.ds(base, _BPW)], idxh_v)
    pltpu.sync_copy(bt_hbm.at[pl.ds(base, _BPW)], idxt_v)
    pltpu.sync_copy(br_hbm.at[pl.ds(base, _BPW)], idxr_v)
    for g in range(_NG):
        cols = pl.ds(g * _L, _L)
        bh0 = idxh_v[cols]
        bt0 = idxt_v[cols]
        br0 = idxr_v[cols]
        for d in range(LATENT):
            fcols = pl.ds(d * _BPW + g * _L, _L)
            fh_v[fcols] = bh0 + d * ENT
            ft_v[fcols] = bt0 + d * ENT
            fr_v[fcols] = br0 + d * REL
    ch = pltpu.async_copy(ent_hbm.at[fh_v], gh_v, sem_h)
    ct = pltpu.async_copy(ent_hbm.at[ft_v], gt_v, sem_t)
    cr = pltpu.async_copy(rel_hbm.at[fr_v], gr_v, sem_r)
    ch.wait()
    ct.wait()
    cr.wait()
    ssq_acc = jnp.zeros((_L,), jnp.float32)
    for g in range(_NG):
        cols = pl.ds(g * _L, _L)

        def body(d, carry):
            acc, sq = carry
            fcols = pl.ds(d * _BPW + g * _L, _L)
            h = gh_v[fcols]
            t = gt_v[fcols]
            r = gr_v[fcols]
            acc = acc + h * t * r
            sq = sq + h * h + t * t + r * r
            return acc, sq

        acc, ssq_acc = lax.fori_loop(
            0, LATENT, body, (jnp.zeros((_L,), jnp.float32), ssq_acc))
        s_v[cols] = acc
    ssq_v[...] = ssq_acc
    pltpu.sync_copy(s_v, s_out.at[pl.ds(base, _BPW)])
    pltpu.sync_copy(ssq_v, ssq_out.at[pl.ds(wid * _L, _L)])


_score = functools.partial(
    pl.kernel,
    out_type=[
        jax.ShapeDtypeStruct((TWOB,), jnp.float32),
        jax.ShapeDtypeStruct((_NW * _L,), jnp.float32),
    ],
    mesh=plsc.VectorSubcoreMesh(core_axis_name="c", subcore_axis_name="s"),
    scratch_types=[
        pltpu.VMEM((_BPW,), jnp.int32),
        pltpu.VMEM((_BPW,), jnp.int32),
        pltpu.VMEM((_BPW,), jnp.int32),
        pltpu.VMEM((LATENT * _BPW,), jnp.int32),
        pltpu.VMEM((LATENT * _BPW,), jnp.int32),
        pltpu.VMEM((LATENT * _BPW,), jnp.int32),
        pltpu.VMEM((LATENT * _BPW,), jnp.float32),
        pltpu.VMEM((LATENT * _BPW,), jnp.float32),
        pltpu.VMEM((LATENT * _BPW,), jnp.float32),

        pltpu.VMEM((_BPW,), jnp.float32),
        pltpu.VMEM((_L,), jnp.float32),
        pltpu.SemaphoreType.DMA,
        pltpu.SemaphoreType.DMA,
        pltpu.SemaphoreType.DMA,
    ],
)(_score_body)


def _loss_body(s_ref, ssq_ref, take2_ref, loss_ref):
    s = s_ref[...]
    a = jnp.abs(s)
    sp_pair = a + 2.0 * jnp.log1p(jnp.exp(-a))
    contrib = jnp.where(take2_ref[...] > 0, sp_pair, 2.0 * _LOG2)
    loss_main = jnp.sum(contrib) / (4.0 * BATCH)
    regul = jnp.sum(ssq_ref[...]) / float(TWOB * LATENT)
    loss_ref[...] = jnp.broadcast_to(loss_main + LMBDA * regul, (1, 1))


def kernel(ent_embeddings, rel_embeddings, pos_h, pos_r, pos_t,
           neg_h, neg_r, neg_t, take):
    bh = jnp.concatenate([pos_h, neg_h]).astype(jnp.int32)
    bt = jnp.concatenate([pos_t, neg_t]).astype(jnp.int32)
    br = jnp.concatenate([pos_r, neg_r]).astype(jnp.int32)
    take2 = jnp.concatenate([take, take]).astype(jnp.float32)
    ent1 = ent_embeddings.T.reshape(-1)
    rel1 = rel_embeddings.T.reshape(-1)

    s, ssq = _score(ent1, rel1, bh, bt, br)

    loss2d = pl.pallas_call(
        _loss_body,
        out_shape=jax.ShapeDtypeStruct((1, 1), jnp.float32),
    )(s, ssq, take2)
    return loss2d[0, 0], s[BATCH:]


# speedup vs baseline: 13.8318x; 13.8318x over previous
import functools

import jax
import jax.numpy as jnp
import numpy as np
from jax import lax
from jax.experimental import pallas as pl
from jax.experimental.pallas import tpu as pltpu
from jax.experimental.pallas import tpu_sc as plsc

LATENT = 64
BATCH = 1024
TWOB = 2 * BATCH
REL = 1000
LMBDA = 0.1
_LOG2 = float(np.log(2.0))

_info = plsc.get_sparse_core_info()
_NC, _NS = _info.num_cores, _info.num_subcores
_NW = _NC * _NS
_BPW = TWOB // _NW


def _gather_body(ent_hbm, bh_hbm, bt_hbm, eh_out, et_out,
                 idxh_v, idxt_v, rh_v, rt_v, sem):
    wid = lax.axis_index("s") * _NC + lax.axis_index("c")
    base = wid * _BPW
    pltpu.sync_copy(bh_hbm.at[pl.ds(base, _BPW)], idxh_v)
    pltpu.sync_copy(bt_hbm.at[pl.ds(base, _BPW)], idxt_v)
    copies = []
    for g in range(_BPW // 16):
        vh = idxh_v[pl.ds(g * 16, 16)]
        vt = idxt_v[pl.ds(g * 16, 16)]
        for l in range(16):
            i = g * 16 + l
            copies.append(pltpu.async_copy(
                ent_hbm.at[pl.ds(vh[l], 1)], rh_v.at[pl.ds(i, 1)], sem))
            copies.append(pltpu.async_copy(
                ent_hbm.at[pl.ds(vt[l], 1)], rt_v.at[pl.ds(i, 1)], sem))
    for c in copies:
        c.wait()
    pltpu.sync_copy(rh_v, eh_out.at[pl.ds(base, _BPW)])
    pltpu.sync_copy(rt_v, et_out.at[pl.ds(base, _BPW)])


_gather2 = functools.partial(
    pl.kernel,
    out_type=[
        jax.ShapeDtypeStruct((TWOB, LATENT), jnp.float32),
        jax.ShapeDtypeStruct((TWOB, LATENT), jnp.float32),
    ],
    mesh=plsc.VectorSubcoreMesh(core_axis_name="c", subcore_axis_name="s"),
    compiler_params=pltpu.CompilerParams(skip_device_barrier=True),
    scratch_types=[
        pltpu.VMEM((_BPW,), jnp.int32),
        pltpu.VMEM((_BPW,), jnp.int32),
        pltpu.VMEM((_BPW, LATENT), jnp.float32),
        pltpu.VMEM((_BPW, LATENT), jnp.float32),
        pltpu.SemaphoreType.DMA,
    ],
)(_gather_body)


def _finish_body(eh_ref, et_ref, rel_ref, br_ref, take2_ref,
                 loss_ref, s_ref):
    eh = eh_ref[...]
    et = et_ref[...]
    rel = rel_ref[...]
    br = br_ref[...]
    onehot = (br[:, None] ==
              lax.broadcasted_iota(jnp.int32, (TWOB, REL), 1)
              ).astype(jnp.float32)
    er = jnp.dot(onehot, rel, preferred_element_type=jnp.float32)
    s = jnp.sum(eh * et * er, axis=1)
    s_ref[...] = s
    a = jnp.abs(s)
    sp_pair = a + 2.0 * jnp.log1p(jnp.exp(-a))
    contrib = jnp.where(take2_ref[...] > 0, sp_pair, 2.0 * _LOG2)
    loss_main = jnp.sum(contrib) / (4.0 * BATCH)
    ssq = jnp.sum(eh * eh) + jnp.sum(et * et) + jnp.sum(er * er)
    regul = ssq / float(TWOB * LATENT)
    loss_ref[...] = jnp.broadcast_to(loss_main + LMBDA * regul, (1, 1))


def kernel(ent_embeddings, rel_embeddings, pos_h, pos_r, pos_t,
           neg_h, neg_r, neg_t, take):
    bh = jnp.concatenate([pos_h, neg_h]).astype(jnp.int32)
    bt = jnp.concatenate([pos_t, neg_t]).astype(jnp.int32)
    br = jnp.concatenate([pos_r, neg_r]).astype(jnp.int32)
    take2 = jnp.concatenate([take, take]).astype(jnp.float32)

    eh, et = _gather2(ent_embeddings, bh, bt)

    loss2d, s = pl.pallas_call(
        _finish_body,
        out_shape=[
            jax.ShapeDtypeStruct((1, 1), jnp.float32),
            jax.ShapeDtypeStruct((TWOB, ), jnp.float32),
        ],
    )(eh, et, rel_embeddings, br, take2)
    return loss2d[0, 0], s[BATCH:]


# speedup vs baseline: 45.1382x; 3.2634x over previous
import functools

import jax
import jax.numpy as jnp
import numpy as np
from jax import lax
from jax.experimental import pallas as pl
from jax.experimental.pallas import tpu as pltpu
from jax.experimental.pallas import tpu_sc as plsc

LATENT = 64
BATCH = 1024
TWOB = 2 * BATCH
REL = 1000
LMBDA = 0.1
_LOG2 = float(np.log(2.0))

_info = plsc.get_sparse_core_info()
_NC, _NS, _L = _info.num_cores, _info.num_subcores, _info.num_lanes
_NW = _NC * _NS
_BPW = TWOB // _NW


def _gather_body(entT_hbm, bh_hbm, bt_hbm, eh_out, et_out,
                 idxh_v, idxt_v, rh_v, rt_v, blk0, blk1, sem0, sem1):
    wid = lax.axis_index("s") * _NC + lax.axis_index("c")
    base = wid * _BPW
    pltpu.sync_copy(bh_hbm.at[pl.ds(base, _BPW)], idxh_v)
    pltpu.sync_copy(bt_hbm.at[pl.ds(base, _BPW)], idxt_v)
    lane = lax.iota(jnp.int32, _L)

    idxs = []
    for src_v in (idxh_v, idxt_v):
        for g in range(_BPW // _L):
            vv = src_v[pl.ds(g * _L, _L)]
            for l in range(_L):
                idxs.append(vv[l])

    bufs = (blk0, blk1)
    sems = (sem0, sem1)

    def fire(i):
        c0 = pl.multiple_of((idxs[i] >> 7) * 128, 128)
        return pltpu.async_copy(
            entT_hbm.at[:, pl.ds(c0, 128)], bufs[i % 2], sems[i % 2])

    pending = {0: fire(0)}
    for i in range(2 * _BPW):
        if i + 1 < 2 * _BPW:
            pending[i + 1] = fire(i + 1)
        pending.pop(i).wait()
        buf = bufs[i % 2]
        col = jnp.full((_L,), 0, jnp.int32) + (idxs[i] & 127)
        dst_v = rh_v if i < _BPW else rt_v
        row = i % _BPW
        for ch in range(LATENT // _L):
            vals = plsc.load_gather(buf, [ch * _L + lane, col])
            dst_v[row, pl.ds(ch * _L, _L)] = vals

    pltpu.sync_copy(rh_v, eh_out.at[pl.ds(base, _BPW)])
    pltpu.sync_copy(rt_v, et_out.at[pl.ds(base, _BPW)])


_gather2 = functools.partial(
    pl.kernel,
    out_type=[
        jax.ShapeDtypeStruct((TWOB, LATENT), jnp.float32),
        jax.ShapeDtypeStruct((TWOB, LATENT), jnp.float32),
    ],
    mesh=plsc.VectorSubcoreMesh(core_axis_name="c", subcore_axis_name="s"),
    compiler_params=pltpu.CompilerParams(needs_layout_passes=False),
    scratch_types=[
        pltpu.VMEM((_BPW,), jnp.int32),
        pltpu.VMEM((_BPW,), jnp.int32),
        pltpu.VMEM((_BPW, LATENT), jnp.float32),
        pltpu.VMEM((_BPW, LATENT), jnp.float32),
        pltpu.VMEM((LATENT, 128), jnp.float32),
        pltpu.VMEM((LATENT, 128), jnp.float32),
        pltpu.SemaphoreType.DMA,
        pltpu.SemaphoreType.DMA,
    ],
)(_gather_body)


def _finish_body(eh_ref, et_ref, rel_ref, br_ref, take2_ref,
                 loss_ref, s_ref):
    eh = eh_ref[...]
    et = et_ref[...]
    rel = rel_ref[...]
    br = br_ref[...]
    onehot = (br[:, None] ==
              lax.broadcasted_iota(jnp.int32, (TWOB, REL), 1)
              ).astype(jnp.float32)
    er = jnp.dot(onehot, rel, preferred_element_type=jnp.float32)
    s = jnp.sum(eh * et * er, axis=1)
    s_ref[...] = s
    a = jnp.abs(s)
    sp_pair = a + 2.0 * jnp.log1p(jnp.exp(-a))
    contrib = jnp.where(take2_ref[...] > 0, sp_pair, 2.0 * _LOG2)
    loss_main = jnp.sum(contrib) / (4.0 * BATCH)
    ssq = jnp.sum(eh * eh) + jnp.sum(et * et) + jnp.sum(er * er)
    regul = ssq / float(TWOB * LATENT)
    loss_ref[...] = jnp.broadcast_to(loss_main + LMBDA * regul, (1, 1))


def kernel(ent_embeddings, rel_embeddings, pos_h, pos_r, pos_t,
           neg_h, neg_r, neg_t, take):
    bh = jnp.concatenate([pos_h, neg_h]).astype(jnp.int32)
    bt = jnp.concatenate([pos_t, neg_t]).astype(jnp.int32)
    br = jnp.concatenate([pos_r, neg_r]).astype(jnp.int32)
    take2 = jnp.concatenate([take, take]).astype(jnp.float32)
    entT = ent_embeddings.T

    eh, et = _gather2(entT, bh, bt)

    loss2d, s = pl.pallas_call(
        _finish_body,
        out_shape=[
            jax.ShapeDtypeStruct((1, 1), jnp.float32),
            jax.ShapeDtypeStruct((TWOB,), jnp.float32),
        ],
    )(eh, et, rel_embeddings, br, take2)
    return loss2d[0, 0], s[BATCH:]


# speedup vs baseline: 55.8714x; 1.2378x over previous
import functools

import jax
import jax.numpy as jnp
import numpy as np
from jax import lax
from jax.experimental import pallas as pl
from jax.experimental.pallas import tpu as pltpu
from jax.experimental.pallas import tpu_sc as plsc

LATENT = 64
BATCH = 1024
TWOB = 2 * BATCH
REL = 1000
LMBDA = 0.1
_LOG2 = float(np.log(2.0))

_info = plsc.get_sparse_core_info()
_NC, _NS, _L = _info.num_cores, _info.num_subcores, _info.num_lanes
_NW = _NC * _NS
_BPW = TWOB // _NW


def _gather_body(entT_hbm, bh_hbm, bt_hbm, eh_out, et_out,
                 idxh_v, idxt_v, rh_v, rt_v, blk0, blk1, blk2, blk3,
                 sem0, sem1, sem2, sem3):
    wid = lax.axis_index("s") * _NC + lax.axis_index("c")
    base = wid * _BPW
    pltpu.sync_copy(bh_hbm.at[pl.ds(base, _BPW)], idxh_v)
    pltpu.sync_copy(bt_hbm.at[pl.ds(base, _BPW)], idxt_v)
    lane = lax.iota(jnp.int32, _L)

    idxs = []
    for src_v in (idxh_v, idxt_v):
        for g in range(_BPW // _L):
            vv = src_v[pl.ds(g * _L, _L)]
            for l in range(_L):
                idxs.append(vv[l])

    bufs = (blk0, blk1, blk2, blk3)
    sems = (sem0, sem1, sem2, sem3)
    nbuf = len(bufs)

    def fire(i):
        c0 = pl.multiple_of((idxs[i] >> 7) * 128, 128)
        return pltpu.async_copy(
            entT_hbm.at[:, pl.ds(c0, 128)], bufs[i % nbuf], sems[i % nbuf])

    pending = {i: fire(i) for i in range(nbuf - 1)}
    for i in range(2 * _BPW):
        if i + nbuf - 1 < 2 * _BPW:
            pending[i + nbuf - 1] = fire(i + nbuf - 1)
        pending.pop(i).wait()
        buf = bufs[i % nbuf]
        col = jnp.full((_L,), 0, jnp.int32) + (idxs[i] & 127)
        dst_v = rh_v if i < _BPW else rt_v
        row = i % _BPW
        for ch in range(LATENT // _L):
            vals = plsc.load_gather(buf, [ch * _L + lane, col])
            dst_v[row, pl.ds(ch * _L, _L)] = vals

    pltpu.sync_copy(rh_v, eh_out.at[pl.ds(base, _BPW)])
    pltpu.sync_copy(rt_v, et_out.at[pl.ds(base, _BPW)])


_gather2 = functools.partial(
    pl.kernel,
    out_type=[
        jax.ShapeDtypeStruct((TWOB, LATENT), jnp.float32),
        jax.ShapeDtypeStruct((TWOB, LATENT), jnp.float32),
    ],
    mesh=plsc.VectorSubcoreMesh(core_axis_name="c", subcore_axis_name="s"),
    compiler_params=pltpu.CompilerParams(needs_layout_passes=False),
    scratch_types=[
        pltpu.VMEM((_BPW,), jnp.int32),
        pltpu.VMEM((_BPW,), jnp.int32),
        pltpu.VMEM((_BPW, LATENT), jnp.float32),
        pltpu.VMEM((_BPW, LATENT), jnp.float32),
        pltpu.VMEM((LATENT, 128), jnp.float32),
        pltpu.VMEM((LATENT, 128), jnp.float32),
        pltpu.VMEM((LATENT, 128), jnp.float32),
        pltpu.VMEM((LATENT, 128), jnp.float32),
        pltpu.SemaphoreType.DMA,
        pltpu.SemaphoreType.DMA,
        pltpu.SemaphoreType.DMA,
        pltpu.SemaphoreType.DMA,
    ],
)(_gather_body)


def _finish_body(eh_ref, et_ref, rel_ref, br_ref, take2_ref,
                 loss_ref, s_ref):
    eh = eh_ref[...]
    et = et_ref[...]
    rel = rel_ref[...]
    br = br_ref[...]
    onehot = (br[:, None] ==
              lax.broadcasted_iota(jnp.int32, (TWOB, REL), 1)
              ).astype(jnp.float32)
    er = jnp.dot(onehot, rel, preferred_element_type=jnp.float32)
    s = jnp.sum(eh * et * er, axis=1)
    s_ref[...] = s
    a = jnp.abs(s)
    sp_pair = a + 2.0 * jnp.log1p(jnp.exp(-a))
    contrib = jnp.where(take2_ref[...] > 0, sp_pair, 2.0 * _LOG2)
    loss_main = jnp.sum(contrib) / (4.0 * BATCH)
    ssq = jnp.sum(eh * eh) + jnp.sum(et * et) + jnp.sum(er * er)
    regul = ssq / float(TWOB * LATENT)
    loss_ref[...] = jnp.broadcast_to(loss_main + LMBDA * regul, (1, 1))


def kernel(ent_embeddings, rel_embeddings, pos_h, pos_r, pos_t,
           neg_h, neg_r, neg_t, take):
    bh = jnp.concatenate([pos_h, neg_h]).astype(jnp.int32)
    bt = jnp.concatenate([pos_t, neg_t]).astype(jnp.int32)
    br = jnp.concatenate([pos_r, neg_r]).astype(jnp.int32)
    take2 = jnp.concatenate([take, take]).astype(jnp.float32)
    entT = ent_embeddings.T

    eh, et = _gather2(entT, bh, bt)

    loss2d, s = pl.pallas_call(
        _finish_body,
        out_shape=[
            jax.ShapeDtypeStruct((1, 1), jnp.float32),
            jax.ShapeDtypeStruct((TWOB,), jnp.float32),
        ],
    )(eh, et, rel_embeddings, br, take2)
    return loss2d[0, 0], s[BATCH:]


# speedup vs baseline: 62.2827x; 1.1148x over previous
import functools

import jax
import jax.numpy as jnp
import numpy as np
from jax import lax
from jax.experimental import pallas as pl
from jax.experimental.pallas import tpu as pltpu
from jax.experimental.pallas import tpu_sc as plsc

LATENT = 64
BATCH = 1024
TWOB = 2 * BATCH
REL = 1000
LMBDA = 0.1
_LOG2 = float(np.log(2.0))

_info = plsc.get_sparse_core_info()
_NC, _NS, _L = _info.num_cores, _info.num_subcores, _info.num_lanes
_NW = _NC * _NS
_BPW = TWOB // _NW


def _gather_body(entT_hbm, bh_hbm, bt_hbm, eh_out, et_out,
                 idxh_v, idxt_v, rh_v, rt_v,
                 blk0, blk1, blk2, blk3, blk4, blk5, blk6, blk7,
                 sem0, sem1, sem2, sem3, sem4, sem5, sem6, sem7):
    wid = lax.axis_index("s") * _NC + lax.axis_index("c")
    base = wid * _BPW
    pltpu.sync_copy(bh_hbm.at[pl.ds(base, _BPW)], idxh_v)
    pltpu.sync_copy(bt_hbm.at[pl.ds(base, _BPW)], idxt_v)
    lane = lax.iota(jnp.int32, _L)

    idxs = []
    for src_v in (idxh_v, idxt_v):
        for g in range(_BPW // _L):
            vv = src_v[pl.ds(g * _L, _L)]
            for l in range(_L):
                idxs.append(vv[l])

    bufs = (blk0, blk1, blk2, blk3, blk4, blk5, blk6, blk7)
    sems = (sem0, sem1, sem2, sem3, sem4, sem5, sem6, sem7)
    nbuf = len(bufs)

    def fire(i):
        c0 = pl.multiple_of((idxs[i] >> 7) * 128, 128)
        return pltpu.async_copy(
            entT_hbm.at[:, pl.ds(c0, 128)], bufs[i % nbuf], sems[i % nbuf])

    pending = {i: fire(i) for i in range(nbuf - 1)}
    for i in range(2 * _BPW):
        if i + nbuf - 1 < 2 * _BPW:
            pending[i + nbuf - 1] = fire(i + nbuf - 1)
        pending.pop(i).wait()
        buf = bufs[i % nbuf]
        col = jnp.full((_L,), 0, jnp.int32) + (idxs[i] & 127)
        dst_v = rh_v if i < _BPW else rt_v
        row = i % _BPW
        for ch in range(LATENT // _L):
            vals = plsc.load_gather(buf, [ch * _L + lane, col])
            dst_v[row, pl.ds(ch * _L, _L)] = vals

    pltpu.sync_copy(rh_v, eh_out.at[pl.ds(base, _BPW)])
    pltpu.sync_copy(rt_v, et_out.at[pl.ds(base, _BPW)])


_gather2 = functools.partial(
    pl.kernel,
    out_type=[
        jax.ShapeDtypeStruct((TWOB, LATENT), jnp.float32),
        jax.ShapeDtypeStruct((TWOB, LATENT), jnp.float32),
    ],
    mesh=plsc.VectorSubcoreMesh(core_axis_name="c", subcore_axis_name="s"),
    compiler_params=pltpu.CompilerParams(needs_layout_passes=False),
    scratch_types=[
        pltpu.VMEM((_BPW,), jnp.int32),
        pltpu.VMEM((_BPW,), jnp.int32),
        pltpu.VMEM((_BPW, LATENT), jnp.float32),
        pltpu.VMEM((_BPW, LATENT), jnp.float32),
        pltpu.VMEM((LATENT, 128), jnp.float32),
        pltpu.VMEM((LATENT, 128), jnp.float32),
        pltpu.VMEM((LATENT, 128), jnp.float32),
        pltpu.VMEM((LATENT, 128), jnp.float32),
        pltpu.VMEM((LATENT, 128), jnp.float32),
        pltpu.VMEM((LATENT, 128), jnp.float32),
        pltpu.VMEM((LATENT, 128), jnp.float32),
        pltpu.VMEM((LATENT, 128), jnp.float32),
        pltpu.SemaphoreType.DMA,
        pltpu.SemaphoreType.DMA,
        pltpu.SemaphoreType.DMA,
        pltpu.SemaphoreType.DMA,
        pltpu.SemaphoreType.DMA,
        pltpu.SemaphoreType.DMA,
        pltpu.SemaphoreType.DMA,
        pltpu.SemaphoreType.DMA,
    ],
)(_gather_body)


def _finish_body(eh_ref, et_ref, rel_ref, br_ref, take2_ref,
                 loss_ref, s_ref):
    eh = eh_ref[...]
    et = et_ref[...]
    rel = rel_ref[...]
    br = br_ref[...]
    onehot = (br[:, None] ==
              lax.broadcasted_iota(jnp.int32, (TWOB, REL), 1)
              ).astype(jnp.float32)
    er = jnp.dot(onehot, rel, preferred_element_type=jnp.float32)
    s = jnp.sum(eh * et * er, axis=1)
    s_ref[...] = s
    a = jnp.abs(s)
    sp_pair = a + 2.0 * jnp.log1p(jnp.exp(-a))
    contrib = jnp.where(take2_ref[...] > 0, sp_pair, 2.0 * _LOG2)
    loss_main = jnp.sum(contrib) / (4.0 * BATCH)
    ssq = jnp.sum(eh * eh) + jnp.sum(et * et) + jnp.sum(er * er)
    regul = ssq / float(TWOB * LATENT)
    loss_ref[...] = jnp.broadcast_to(loss_main + LMBDA * regul, (1, 1))


def kernel(ent_embeddings, rel_embeddings, pos_h, pos_r, pos_t,
           neg_h, neg_r, neg_t, take):
    bh = jnp.concatenate([pos_h, neg_h]).astype(jnp.int32)
    bt = jnp.concatenate([pos_t, neg_t]).astype(jnp.int32)
    br = jnp.concatenate([pos_r, neg_r]).astype(jnp.int32)
    take2 = jnp.concatenate([take, take]).astype(jnp.float32)
    entT = ent_embeddings.T

    eh, et = _gather2(entT, bh, bt)

    loss2d, s = pl.pallas_call(
        _finish_body,
        out_shape=[
            jax.ShapeDtypeStruct((1, 1), jnp.float32),
            jax.ShapeDtypeStruct((TWOB,), jnp.float32),
        ],
    )(eh, et, rel_embeddings, br, take2)
    return loss2d[0, 0], s[BATCH:]
